# baseline (speedup 1.0000x reference)
import functools
import jax
import jax.numpy as jnp
from jax import lax
from jax.experimental import pallas as pl
from jax.experimental.pallas import tpu as pltpu
from jax.experimental.pallas import tpu_sc as plsc

N = 10000
E = 320000
DA = 16
DH = 128
NG = 64
NC = 2
NS = 16
NW = NC * NS
EPW = E // NW
CH = 80
NCHUNK = EPW // CH
RPS = 640
RPS_LAST = N - RPS * (NS - 1)

_mesh = plsc.VectorSubcoreMesh(core_axis_name="c", subcore_axis_name="s")


def _worker_id():
    return lax.axis_index("s") * NC + lax.axis_index("c")


def _rowwise(s, fn):
    @pl.when(s < NS - 1)
    def _():
        fn(s * RPS, RPS)

    @pl.when(s == NS - 1)
    def _():
        fn((NS - 1) * RPS, RPS_LAST)


def _make_seg(mode, ch):
    weighted = mode == "weighted"
    hopfused = mode == "hopfused"
    nchunk = EPW // ch
    scratch = [
        pltpu.VMEM((ch,), jnp.int32),
        pltpu.VMEM((ch,), jnp.int32),
        pltpu.VMEM((ch, DH), jnp.float32),
        pltpu.VMEM((ch, DH), jnp.float32),
        pltpu.VMEM_SHARED((N, DH), jnp.float32),
        pltpu.SemaphoreType.DMA,
        pltpu.SemaphoreType.DMA,
        pltpu.SemaphoreType.DMA,
        pltpu.SemaphoreType.DMA,
        pltpu.VMEM((EPW,), jnp.int32),
    ]
    if weighted:
        scratch += [
            pltpu.VMEM((ch * 16,), jnp.float32),
            pltpu.VMEM((ch * 16,), jnp.float32),
        ]
    else:
        scratch += [
            pltpu.VMEM((ch, DH), jnp.float32),
            pltpu.VMEM((ch, DH), jnp.float32),
        ]
    scratch += [
        pltpu.SemaphoreType.DMA,
        pltpu.SemaphoreType.DMA,
    ]

    @functools.partial(
        pl.kernel,
        out_type=jax.ShapeDtypeStruct((NC, N, DH), jnp.float32),
        mesh=_mesh,
        scratch_types=scratch,
    )
    def seg(x, aux, src1, dst1, zeros, out,
            didx0, didx1, rows0, rows1, acc, gs0, gs1, ds0, ds1,
            sidx_all, ab0, ab1, ws0, ws1):
        c = lax.axis_index("c")
        s = lax.axis_index("s")
        wid = _worker_id()
        base = wid * EPW

        _rowwise(s, lambda row0, nrows: pltpu.sync_copy(
            zeros.at[pl.ds(0, nrows)], acc.at[pl.ds(row0, nrows)]))
        pltpu.sync_copy(src1.at[pl.ds(base, EPW)], sidx_all)
        plsc.subcore_barrier()

        def start_load(j, rows_n, gsem_n, didx_n, dsem_n, ab_n, wsem_n):
            pltpu.async_copy(dst1.at[pl.ds(base + j * ch, ch)], didx_n,
                             dsem_n)
            pltpu.async_copy(x.at[sidx_all.at[pl.ds(j * ch, ch)]],
                             rows_n, gsem_n)
            if weighted:
                pltpu.async_copy(
                    aux.at[pl.ds((base + j * ch) * 16, ch * 16)], ab_n,
                    wsem_n)
            else:
                pltpu.async_copy(aux.at[pl.ds(base + j * ch, ch)], ab_n,
                                 wsem_n)

        def step(j, rows_c, gsem_c, didx_c, dsem_c, ab_c, wsem_c, nxt):
            if nxt is not None:
                start_load(j + 1, *nxt)
            pltpu.make_async_copy(x.at[pl.ds(0, ch)], rows_c, gsem_c).wait()
            pltpu.make_async_copy(dst1.at[pl.ds(0, ch)], didx_c,
                                  dsem_c).wait()
            if weighted:
                pltpu.make_async_copy(aux.at[pl.ds(0, ch * 16)], ab_c,
                                      wsem_c).wait()

                def scale(r, cc):
                    wv = ab_c[pl.ds(r * 16, 16)]
                    for g in range(DH // 16):
                        sl = pl.ds(g * 16, 16)
                        rows_c[r, sl] = rows_c[r, sl] * wv
                    return cc

                lax.fori_loop(0, ch, scale, 0)
            else:
                pltpu.make_async_copy(x.at[pl.ds(0, ch)], ab_c,
                                      wsem_c).wait()

                def fuse(r, cc):
                    for g in range(DH // 16):
                        sl = pl.ds(g * 16, 16)
                        rows_c[r, sl] = rows_c[r, sl] + ab_c[r, sl]
                    return cc

                lax.fori_loop(0, ch, fuse, 0)
            pltpu.sync_copy(rows_c, acc.at[didx_c], add=True)

        b0 = (rows0, gs0, didx0, ds0, ab0, ws0)
        b1 = (rows1, gs1, didx1, ds1, ab1, ws1)
        start_load(0, *b0)

        def pair(i, carry):
            step(2 * i, *b0, nxt=b1)
            step(2 * i + 1, *b1, nxt=b0)
            return carry

        npair = (nchunk - 1) // 2
        lax.fori_loop(0, npair, pair, 0)
        if nchunk - 2 * npair == 2:
            step(nchunk - 2, *b0, nxt=b1)
            step(nchunk - 1, *b1, nxt=None)
        else:
            step(nchunk - 1, *b0, nxt=None)

        plsc.subcore_barrier()
        _rowwise(s, lambda row0, nrows: pltpu.sync_copy(
            acc.at[pl.ds(row0, nrows)], out.at[c, pl.ds(row0, nrows)]))

    return seg


_seg128_hopfused_k = _make_seg("hopfused", 40)
_seg128_weighted_k = _make_seg("weighted", 80)


_NB = 10
_BR = N // _NB


def _mm_body(x_ref, W_ref, o_ref):
    o_ref[...] = jnp.dot(x_ref[...], W_ref[...],
                         preferred_element_type=jnp.float32)


def _t1_body(h_ref, a_ref, Wm_ref, bm_ref, out_ref):
    pre = h_ref[...] + a_ref[0] + a_ref[1]
    out_ref[...] = jnp.maximum(
        jnp.dot(pre, Wm_ref[...], preferred_element_type=jnp.float32)
        + bm_ref[...], 0.0)


def _t2_body(h1_ref, a2_ref, Wm_ref, bm_ref, Wg1_ref, bg1_ref,
             x1_ref):
    pre = h1_ref[...] + a2_ref[0] + a2_ref[1]
    h2 = jnp.maximum(
        jnp.dot(pre, Wm_ref[...], preferred_element_type=jnp.float32)
        + bm_ref[...], 0.0)
    x1_ref[...] = jnp.dot(h2, Wg1_ref[...],
                          preferred_element_type=jnp.float32) + bg1_ref[...]


def _t3_body(s1_ref, Wg2_ref, bg2_ref, x2_ref):
    h3 = jnp.maximum(s1_ref[0] + s1_ref[1], 0.0)
    x2_ref[...] = jnp.dot(h3, Wg2_ref[...],
                          preferred_element_type=jnp.float32) + bg2_ref[...]


def _t4_body(s2_ref, batch_ref, Wp1_ref, bp1_ref, Wp2_ref, bp2_ref,
             y_ref, fp_ref):
    i = pl.program_id(0)

    @pl.when(i == 0)
    def _():
        fp_ref[...] = jnp.zeros_like(fp_ref)

    h4 = jnp.maximum(s2_ref[0] + s2_ref[1], 0.0)
    b = batch_ref[0]
    onehot = (lax.broadcasted_iota(jnp.int32, (NG, _BR), 0) == b).astype(
        jnp.float32)
    fp_ref[...] += jnp.dot(onehot, h4, preferred_element_type=jnp.float32)

    @pl.when(i == _NB - 1)
    def _():
        hidden = jnp.maximum(
            jnp.dot(fp_ref[...], Wp1_ref[...],
                    preferred_element_type=jnp.float32) + bp1_ref[...], 0.0)
        y_ref[...] = jnp.dot(hidden, Wp2_ref[...],
                             preferred_element_type=jnp.float32) + bp2_ref[...]


def _row_spec(d):
    return pl.BlockSpec((_BR, d), lambda i: (i, 0))


def _part_spec(d):
    return pl.BlockSpec((NC, _BR, d), lambda i: (0, i, 0))


def _full_spec(a, b):
    return pl.BlockSpec((a, b), lambda i: (0, 0))


def kernel(node_attr, edge_attr, edge_index, adj_index, adj_value, batch,
           num_nodes, Wn, We, Wm, bm, Wg1, bg1, Wg2, bg2, Wp1, bp1, Wp2, bp2):
    src1 = edge_index[0]
    dst1 = edge_index[1]
    a_src1 = adj_index[0]
    a_dst1 = adj_index[1]
    z128 = jnp.zeros((RPS, DH), jnp.float32)
    wrep = jnp.broadcast_to(adj_value[:, None], (E, 16)).reshape(E * 16)

    bm2 = bm.reshape(1, DH)
    bg1_2 = bg1.reshape(1, DH)
    bg2_2 = bg2.reshape(1, DH)
    bp1_2 = bp1.reshape(1, 64)
    bp2_2 = bp2.reshape(1, 1)

    _EB = E // 160
    h_edge = pl.pallas_call(
        _mm_body,
        grid=(160,),
        in_specs=[pl.BlockSpec((_EB, DA), lambda i: (i, 0)),
                  _full_spec(DA, DH)],
        out_specs=pl.BlockSpec((_EB, DH), lambda i: (i, 0)),
        out_shape=jax.ShapeDtypeStruct((E, DH), jnp.float32),
    )(edge_attr, We)

    h0 = pl.pallas_call(
        _mm_body,
        grid=(_NB,),
        in_specs=[_row_spec(DA), _full_spec(DA, DH)],
        out_specs=_row_spec(DH),
        out_shape=jax.ShapeDtypeStruct((N, DH), jnp.float32),
    )(node_attr, Wn)

    a1 = _seg128_hopfused_k(h0, h_edge, src1, dst1, z128)

    h1 = pl.pallas_call(
        _t1_body,
        grid=(_NB,),
        in_specs=[_row_spec(DH), _part_spec(DH),
                  _full_spec(DH, DH), _full_spec(1, DH)],
        out_specs=_row_spec(DH),
        out_shape=jax.ShapeDtypeStruct((N, DH), jnp.float32),
    )(h0, a1, Wm, bm2)

    a2 = _seg128_hopfused_k(h1, h_edge, src1, dst1, z128)

    x1 = pl.pallas_call(
        _t2_body,
        grid=(_NB,),
        in_specs=[_row_spec(DH), _part_spec(DH),
                  _full_spec(DH, DH), _full_spec(1, DH),
                  _full_spec(DH, DH), _full_spec(1, DH)],
        out_specs=_row_spec(DH),
        out_shape=jax.ShapeDtypeStruct((N, DH), jnp.float32),
    )(h1, a2, Wm, bm2, Wg1, bg1_2)

    s1 = _seg128_weighted_k(x1, wrep, a_src1, a_dst1, z128)

    x2 = pl.pallas_call(
        _t3_body,
        grid=(_NB,),
        in_specs=[_part_spec(DH), _full_spec(DH, DH), _full_spec(1, DH)],
        out_specs=_row_spec(DH),
        out_shape=jax.ShapeDtypeStruct((N, DH), jnp.float32),
    )(s1, Wg2, bg2_2)

    s2 = _seg128_weighted_k(x2, wrep, a_src1, a_dst1, z128)

    batch3 = batch.reshape(_NB, 1, _BR)

    y = pl.pallas_call(
        _t4_body,
        grid=(_NB,),
        in_specs=[_part_spec(DH),
                  pl.BlockSpec((1, 1, _BR), lambda i: (i, 0, 0)),
                  _full_spec(DH, 64), _full_spec(1, 64),
                  _full_spec(64, 1), _full_spec(1, 1)],
        out_specs=pl.BlockSpec((NG, 1), lambda i: (0, 0)),
        out_shape=jax.ShapeDtypeStruct((NG, 1), jnp.float32),
        scratch_shapes=[pltpu.VMEM((NG, DH), jnp.float32)],
    )(s2, batch3, Wp1, bp1_2, Wp2, bp2_2)

    return y


# speedup vs baseline: 1.1087x; 1.1087x over previous
import functools
import jax
import jax.numpy as jnp
from jax import lax
from jax.experimental import pallas as pl
from jax.experimental.pallas import tpu as pltpu
from jax.experimental.pallas import tpu_sc as plsc

N = 10000
E = 320000
DA = 16
DH = 128
NG = 64
NC = 2
NS = 16
NW = NC * NS
EPW = E // NW
CH = 80
NCHUNK = EPW // CH
RPS = 640
RPS_LAST = N - RPS * (NS - 1)

_mesh = plsc.VectorSubcoreMesh(core_axis_name="c", subcore_axis_name="s")


def _worker_id():
    return lax.axis_index("s") * NC + lax.axis_index("c")


def _rowwise(s, fn):
    @pl.when(s < NS - 1)
    def _():
        fn(s * RPS, RPS)

    @pl.when(s == NS - 1)
    def _():
        fn((NS - 1) * RPS, RPS_LAST)


def _make_seg(mode):
    gather = mode in ("plain", "weighted")
    weighted = mode == "weighted"
    nbuf = 3
    scratch = (
        [pltpu.VMEM((CH,), jnp.int32)] * nbuf
        + [pltpu.VMEM((CH, DH), jnp.float32)] * nbuf
        + [pltpu.VMEM_SHARED((N, DH), jnp.float32)]
        + [pltpu.SemaphoreType.DMA] * (2 * nbuf)
    )
    if gather:
        scratch.append(pltpu.VMEM((EPW,), jnp.int32))
    if weighted:
        scratch += (
            [pltpu.VMEM((CH * 16,), jnp.float32)] * nbuf
            + [pltpu.SemaphoreType.DMA] * nbuf
        )

    @functools.partial(
        pl.kernel,
        out_type=jax.ShapeDtypeStruct((NC, N, DH), jnp.float32),
        mesh=_mesh,
        scratch_types=scratch,
    )
    def seg(x, src1, dst1, wvals, zeros, out, *refs):
        didxs, refs = refs[:nbuf], refs[nbuf:]
        rowss, refs = refs[:nbuf], refs[nbuf:]
        acc, refs = refs[0], refs[1:]
        gsems, refs = refs[:nbuf], refs[nbuf:]
        dsems, refs = refs[:nbuf], refs[nbuf:]
        if gather:
            sidx_all, refs = refs[0], refs[1:]
        if weighted:
            wbufs, wsems = refs[:nbuf], refs[nbuf:]
        else:
            wbufs = wsems = (None,) * nbuf
        c = lax.axis_index("c")
        s = lax.axis_index("s")
        wid = _worker_id()
        base = wid * EPW

        _rowwise(s, lambda row0, nrows: pltpu.sync_copy(
            zeros.at[pl.ds(0, nrows)], acc.at[pl.ds(row0, nrows)]))
        if gather:
            pltpu.sync_copy(src1.at[pl.ds(base, EPW)], sidx_all)
        plsc.subcore_barrier()

        def start_load(j, rows_n, gsem_n, didx_n, dsem_n, wb_n, wsem_n):
            pltpu.async_copy(dst1.at[pl.ds(base + j * CH, CH)], didx_n,
                             dsem_n)
            if gather:
                pltpu.async_copy(x.at[sidx_all.at[pl.ds(j * CH, CH)]],
                                 rows_n, gsem_n)
            else:
                pltpu.async_copy(x.at[pl.ds(base + j * CH, CH)], rows_n,
                                 gsem_n)
            if weighted:
                pltpu.async_copy(
                    wvals.at[pl.ds((base + j * CH) * 16, CH * 16)], wb_n,
                    wsem_n)

        def step(j, rows_c, gsem_c, didx_c, dsem_c, wb_c, wsem_c, nxt):
            if nxt is not None:
                start_load(j + nbuf - 1, *nxt)
            pltpu.make_async_copy(x.at[pl.ds(0, CH)], rows_c, gsem_c).wait()
            pltpu.make_async_copy(dst1.at[pl.ds(0, CH)], didx_c,
                                  dsem_c).wait()
            if weighted:
                pltpu.make_async_copy(wvals.at[pl.ds(0, CH * 16)], wb_c,
                                      wsem_c).wait()

                def scale(r, cc):
                    wv = wb_c[pl.ds(r * 16, 16)]
                    for g in range(DH // 16):
                        sl = pl.ds(g * 16, 16)
                        rows_c[r, sl] = rows_c[r, sl] * wv
                    return cc

                lax.fori_loop(0, CH, scale, 0)
            pltpu.sync_copy(rows_c, acc.at[didx_c], add=True)

        bufs = [(rowss[b], gsems[b], didxs[b], dsems[b], wbufs[b], wsems[b])
                for b in range(nbuf)]
        start_load(0, *bufs[0])
        start_load(1, *bufs[1])

        ntrip = (NCHUNK - (nbuf - 1)) // nbuf
        assert NCHUNK == ntrip * nbuf + nbuf - 1

        def trip(i, carry):
            for b in range(nbuf):
                step(nbuf * i + b, *bufs[b], nxt=bufs[(b + 2) % nbuf])
            return carry

        lax.fori_loop(0, ntrip, trip, 0)
        for b in range(nbuf - 1):
            step(ntrip * nbuf + b, *bufs[b], nxt=None)

        plsc.subcore_barrier()
        _rowwise(s, lambda row0, nrows: pltpu.sync_copy(
            acc.at[pl.ds(row0, nrows)], out.at[c, pl.ds(row0, nrows)]))

    return seg


_seg128_linear_k = _make_seg("linear")
_seg128_plain_k = _make_seg("plain")
_seg128_weighted_k = _make_seg("weighted")


_NB = 10
_BR = N // _NB


def _mm_body(x_ref, W_ref, o_ref):
    o_ref[...] = jnp.dot(x_ref[...], W_ref[...],
                         preferred_element_type=jnp.float32)


def _t1_body(h_ref, a_ref, ew_ref, Wm_ref, bm_ref, out_ref):
    pre = h_ref[...] + a_ref[0] + a_ref[1] + ew_ref[0] + ew_ref[1]
    out_ref[...] = jnp.maximum(
        jnp.dot(pre, Wm_ref[...], preferred_element_type=jnp.float32)
        + bm_ref[...], 0.0)


def _t2_body(h1_ref, a2_ref, ew_ref, Wm_ref, bm_ref, Wg1_ref, bg1_ref,
             x1_ref):
    pre = h1_ref[...] + a2_ref[0] + a2_ref[1] + ew_ref[0] + ew_ref[1]
    h2 = jnp.maximum(
        jnp.dot(pre, Wm_ref[...], preferred_element_type=jnp.float32)
        + bm_ref[...], 0.0)
    x1_ref[...] = jnp.dot(h2, Wg1_ref[...],
                          preferred_element_type=jnp.float32) + bg1_ref[...]


def _t3_body(s1_ref, Wg2_ref, bg2_ref, x2_ref):
    h3 = jnp.maximum(s1_ref[0] + s1_ref[1], 0.0)
    x2_ref[...] = jnp.dot(h3, Wg2_ref[...],
                          preferred_element_type=jnp.float32) + bg2_ref[...]


def _t4_body(s2_ref, batch_ref, Wp1_ref, bp1_ref, Wp2_ref, bp2_ref,
             y_ref, fp_ref):
    i = pl.program_id(0)

    @pl.when(i == 0)
    def _():
        fp_ref[...] = jnp.zeros_like(fp_ref)

    h4 = jnp.maximum(s2_ref[0] + s2_ref[1], 0.0)
    b = batch_ref[0]
    onehot = (lax.broadcasted_iota(jnp.int32, (NG, _BR), 0) == b).astype(
        jnp.float32)
    fp_ref[...] += jnp.dot(onehot, h4, preferred_element_type=jnp.float32)

    @pl.when(i == _NB - 1)
    def _():
        hidden = jnp.maximum(
            jnp.dot(fp_ref[...], Wp1_ref[...],
                    preferred_element_type=jnp.float32) + bp1_ref[...], 0.0)
        y_ref[...] = jnp.dot(hidden, Wp2_ref[...],
                             preferred_element_type=jnp.float32) + bp2_ref[...]


def _row_spec(d):
    return pl.BlockSpec((_BR, d), lambda i: (i, 0))


def _part_spec(d):
    return pl.BlockSpec((NC, _BR, d), lambda i: (0, i, 0))


def _full_spec(a, b):
    return pl.BlockSpec((a, b), lambda i: (0, 0))


def kernel(node_attr, edge_attr, edge_index, adj_index, adj_value, batch,
           num_nodes, Wn, We, Wm, bm, Wg1, bg1, Wg2, bg2, Wp1, bp1, Wp2, bp2):
    src1 = edge_index[0]
    dst1 = edge_index[1]
    a_src1 = adj_index[0]
    a_dst1 = adj_index[1]
    z128 = jnp.zeros((RPS, DH), jnp.float32)
    wrep = jnp.broadcast_to(adj_value[:, None], (E, 16)).reshape(E * 16)
    dummy_w = jnp.zeros((128,), jnp.float32)

    bm2 = bm.reshape(1, DH)
    bg1_2 = bg1.reshape(1, DH)
    bg2_2 = bg2.reshape(1, DH)
    bp1_2 = bp1.reshape(1, 64)
    bp2_2 = bp2.reshape(1, 1)

    _EB = E // 160
    h_edge = pl.pallas_call(
        _mm_body,
        grid=(160,),
        in_specs=[pl.BlockSpec((_EB, DA), lambda i: (i, 0)),
                  _full_spec(DA, DH)],
        out_specs=pl.BlockSpec((_EB, DH), lambda i: (i, 0)),
        out_shape=jax.ShapeDtypeStruct((E, DH), jnp.float32),
    )(edge_attr, We)

    ew = _seg128_linear_k(h_edge, dst1, dst1, dummy_w, z128)

    h0 = pl.pallas_call(
        _mm_body,
        grid=(_NB,),
        in_specs=[_row_spec(DA), _full_spec(DA, DH)],
        out_specs=_row_spec(DH),
        out_shape=jax.ShapeDtypeStruct((N, DH), jnp.float32),
    )(node_attr, Wn)

    a1 = _seg128_plain_k(h0, src1, dst1, dummy_w, z128)

    h1 = pl.pallas_call(
        _t1_body,
        grid=(_NB,),
        in_specs=[_row_spec(DH), _part_spec(DH), _part_spec(DH),
                  _full_spec(DH, DH), _full_spec(1, DH)],
        out_specs=_row_spec(DH),
        out_shape=jax.ShapeDtypeStruct((N, DH), jnp.float32),
    )(h0, a1, ew, Wm, bm2)

    a2 = _seg128_plain_k(h1, src1, dst1, dummy_w, z128)

    x1 = pl.pallas_call(
        _t2_body,
        grid=(_NB,),
        in_specs=[_row_spec(DH), _part_spec(DH), _part_spec(DH),
                  _full_spec(DH, DH), _full_spec(1, DH),
                  _full_spec(DH, DH), _full_spec(1, DH)],
        out_specs=_row_spec(DH),
        out_shape=jax.ShapeDtypeStruct((N, DH), jnp.float32),
    )(h1, a2, ew, Wm, bm2, Wg1, bg1_2)

    s1 = _seg128_weighted_k(x1, a_src1, a_dst1, wrep, z128)

    x2 = pl.pallas_call(
        _t3_body,
        grid=(_NB,),
        in_specs=[_part_spec(DH), _full_spec(DH, DH), _full_spec(1, DH)],
        out_specs=_row_spec(DH),
        out_shape=jax.ShapeDtypeStruct((N, DH), jnp.float32),
    )(s1, Wg2, bg2_2)

    s2 = _seg128_weighted_k(x2, a_src1, a_dst1, wrep, z128)

    batch3 = batch.reshape(_NB, 1, _BR)

    y = pl.pallas_call(
        _t4_body,
        grid=(_NB,),
        in_specs=[_part_spec(DH),
                  pl.BlockSpec((1, 1, _BR), lambda i: (i, 0, 0)),
                  _full_spec(DH, 64), _full_spec(1, 64),
                  _full_spec(64, 1), _full_spec(1, 1)],
        out_specs=pl.BlockSpec((NG, 1), lambda i: (0, 0)),
        out_shape=jax.ShapeDtypeStruct((NG, 1), jnp.float32),
        scratch_shapes=[pltpu.VMEM((NG, DH), jnp.float32)],
    )(s2, batch3, Wp1, bp1_2, Wp2, bp2_2)

    return y
